# baseline (speedup 1.0000x reference)
import jax
import jax.numpy as jnp
from jax.experimental import pallas as pl

ALPHA = 0.25
GAMMA = 2.0

_ROWS = 50000
_COLS = 640
_BLOCK_ROWS = 2000


def _focal_block(pred_ref, target_ref, out_ref):
    x = pred_ref[...]
    t = target_ref[...]
    p = jax.nn.sigmoid(x)
    ce = jnp.maximum(x, 0.0) - x * t + jnp.log1p(jnp.exp(-jnp.abs(x)))
    alpha_t = t * ALPHA + (1.0 - t) * (1.0 - ALPHA)
    pt = jnp.where(t == 1.0, p, 1.0 - p)
    one_minus_pt = 1.0 - pt
    out_ref[...] = alpha_t * one_minus_pt * one_minus_pt * ce


def kernel(pred, target):
    shape = pred.shape
    pred2 = pred.reshape(_ROWS, _COLS)
    target2 = target.reshape(_ROWS, _COLS)
    grid = _ROWS // _BLOCK_ROWS
    spec = pl.BlockSpec((_BLOCK_ROWS, _COLS), lambda i: (i, 0))
    out = pl.pallas_call(
        _focal_block,
        grid=(grid,),
        in_specs=[spec, spec],
        out_specs=spec,
        out_shape=jax.ShapeDtypeStruct((_ROWS, _COLS), jnp.float32),
    )(pred2, target2)
    return out.reshape(shape)


# speedup vs baseline: 1.1910x; 1.1910x over previous
import jax
import jax.numpy as jnp
from jax.experimental import pallas as pl

ALPHA = 0.25
GAMMA = 2.0

_ROWS = 400000
_COLS = 80
_BLOCK_ROWS = 8000


def _focal_block(pred_ref, target_ref, out_ref):
    x = pred_ref[...]
    t = target_ref[...]
    p = jax.nn.sigmoid(x)
    ce = jnp.maximum(x, 0.0) - x * t + jnp.log1p(jnp.exp(-jnp.abs(x)))
    alpha_t = t * ALPHA + (1.0 - t) * (1.0 - ALPHA)
    pt = jnp.where(t == 1.0, p, 1.0 - p)
    one_minus_pt = 1.0 - pt
    out_ref[...] = alpha_t * one_minus_pt * one_minus_pt * ce


def kernel(pred, target):
    shape = pred.shape
    pred2 = pred.reshape(_ROWS, _COLS)
    target2 = target.reshape(_ROWS, _COLS)
    grid = _ROWS // _BLOCK_ROWS
    spec = pl.BlockSpec((_BLOCK_ROWS, _COLS), lambda i: (i, 0))
    out = pl.pallas_call(
        _focal_block,
        grid=(grid,),
        in_specs=[spec, spec],
        out_specs=spec,
        out_shape=jax.ShapeDtypeStruct((_ROWS, _COLS), jnp.float32),
    )(pred2, target2)
    return out.reshape(shape)


# speedup vs baseline: 2.4295x; 2.0399x over previous
import jax
import jax.numpy as jnp
from jax.experimental import pallas as pl

ALPHA = 0.25
GAMMA = 2.0

_B = 4
_N = 100000
_C = 80
_BLOCK_N = 4000


def _focal_block(pred_ref, target_ref, out_ref):
    x = pred_ref[...]
    t = target_ref[...]
    p = jax.nn.sigmoid(x)
    ce = jnp.maximum(x, 0.0) - x * t + jnp.log1p(jnp.exp(-jnp.abs(x)))
    alpha_t = t * ALPHA + (1.0 - t) * (1.0 - ALPHA)
    pt = jnp.where(t == 1.0, p, 1.0 - p)
    one_minus_pt = 1.0 - pt
    out_ref[...] = alpha_t * one_minus_pt * one_minus_pt * ce


def kernel(pred, target):
    spec = pl.BlockSpec((1, _BLOCK_N, _C), lambda b, i: (b, i, 0))
    return pl.pallas_call(
        _focal_block,
        grid=(_B, _N // _BLOCK_N),
        in_specs=[spec, spec],
        out_specs=spec,
        out_shape=jax.ShapeDtypeStruct((_B, _N, _C), jnp.float32),
    )(pred, target)


# speedup vs baseline: 2.6290x; 1.0821x over previous
import jax
import jax.numpy as jnp
from jax.experimental import pallas as pl

_B = 4
_N = 100000
_C = 80
_BLOCK_N = 10000


def _focal_block(pred_ref, target_ref, out_ref):
    x = pred_ref[...]
    t = target_ref[...]
    y = x - (t + t) * x
    a = jnp.abs(y)
    z = jnp.exp(-a)
    u = 1.0 + z
    r = 1.0 / u
    sg = jnp.where(y > 0.0, r, z * r)
    sp = jnp.maximum(y, 0.0) + jnp.log(u)
    alpha_t = 0.75 - 0.5 * t
    out_ref[...] = (alpha_t * sp) * (sg * sg)


def kernel(pred, target):
    spec = pl.BlockSpec((1, _BLOCK_N, _C), lambda b, i: (b, i, 0))
    return pl.pallas_call(
        _focal_block,
        grid=(_B, _N // _BLOCK_N),
        in_specs=[spec, spec],
        out_specs=spec,
        out_shape=jax.ShapeDtypeStruct((_B, _N, _C), jnp.float32),
    )(pred, target)


# speedup vs baseline: 12.6420x; 4.8087x over previous
import jax
import jax.numpy as jnp
from jax.experimental import pallas as pl

_B = 4
_N = 100000
_C = 80
_BLOCK_L = 12800


def _focal_block(pred_ref, target_ref, out_ref):
    x = pred_ref[...]
    t = target_ref[...]
    y = x - (t + t) * x
    a = jnp.abs(y)
    z = jnp.exp(-a)
    u = 1.0 + z
    r = 1.0 / u
    sg = jnp.where(y > 0.0, r, z * r)
    sp = jnp.maximum(y, 0.0) + jnp.log(u)
    alpha_t = 0.75 - 0.5 * t
    out_ref[...] = (alpha_t * sp) * (sg * sg)


def kernel(pred, target):
    pred_t = jnp.transpose(pred, (0, 2, 1))
    target_t = jnp.transpose(target, (0, 2, 1))
    grid = (_B, pl.cdiv(_N, _BLOCK_L))
    spec = pl.BlockSpec((1, _C, _BLOCK_L), lambda b, i: (b, 0, i))
    out_t = pl.pallas_call(
        _focal_block,
        grid=grid,
        in_specs=[spec, spec],
        out_specs=spec,
        out_shape=jax.ShapeDtypeStruct((_B, _C, _N), jnp.float32),
    )(pred_t, target_t)
    return jnp.transpose(out_t, (0, 2, 1))
